# baseline (speedup 1.0000x reference)
import functools

import jax
import jax.numpy as jnp
from jax import lax
from jax.experimental import pallas as pl
from jax.experimental.pallas import tpu as pltpu
from jax.experimental.pallas import tpu_sc as plsc

H = 768
EH = 3072
E = 8
S = 2048
T = 256
NTILES = (2 * S) // T + E
GPAD = NTILES * T
Z_COEF = 0.001
LB_COEF = 0.01

_NC, _NS, _NW = 2, 16, 32


def _sc_mesh():
    return plsc.VectorSubcoreMesh(core_axis_name="c", subcore_axis_name="s")

_BF = jnp.bfloat16
_F32 = jnp.float32
_I32 = jnp.int32


def _gelu(x):
    return 0.5 * x * (1.0 + lax.erf(x * (2.0 ** -0.5)))


def _ln(x, g, b):
    m = jnp.mean(x, axis=-1, keepdims=True)
    v = jnp.mean((x - m) ** 2, axis=-1, keepdims=True)
    return (x - m) / jnp.sqrt(v + 1e-5) * g + b


def _router_body(x_ref, g_ref, b_ref, rw_ref, xln_ref, idx_ref, rnk_ref,
                 wgt_ref, cnt_ref, z_ref, lb_ref):
    x = x_ref[...]
    m = jnp.mean(x, axis=-1, keepdims=True)
    v = jnp.mean((x - m) ** 2, axis=-1, keepdims=True)
    xln = (x - m) / jnp.sqrt(v + 1e-5) * g_ref[...] + b_ref[...]
    xln_ref[...] = xln

    logits = jnp.dot(xln, rw_ref[...], preferred_element_type=_F32)
    mx = jnp.max(logits, axis=-1, keepdims=True)
    ex = jnp.exp(logits - mx)
    se = jnp.sum(ex, axis=-1, keepdims=True)
    lse = jnp.log(se) + mx
    z_ref[...] = jnp.mean(lse * lse).reshape(1, 1) * Z_COEF

    p = ex / se
    usage = jnp.mean(p, axis=0, keepdims=True)
    tgt = 1.0 / E
    lb_ref[...] = (jnp.sum(tgt * jnp.log(tgt / usage)) * LB_COEF).reshape(1, 1)

    eid = lax.broadcasted_iota(_I32, (S, E), 1)
    p1 = jnp.max(p, axis=-1, keepdims=True)
    a1 = jnp.min(jnp.where(p == p1, eid, E), axis=-1, keepdims=True)
    pm = jnp.where(eid == a1, -jnp.inf, p)
    p2 = jnp.max(pm, axis=-1, keepdims=True)
    a2 = jnp.min(jnp.where(pm == p2, eid, E), axis=-1, keepdims=True)
    s12 = p1 + p2
    wgt_ref[:, 0:1] = p1 / s12
    wgt_ref[:, 1:2] = p2 / s12
    idx_ref[:, 0:1] = a1
    idx_ref[:, 1:2] = a2

    mask = (eid == a1).astype(_F32) + (eid == a2).astype(_F32)
    ii = lax.broadcasted_iota(_I32, (S, S), 0)
    jj = lax.broadcasted_iota(_I32, (S, S), 1)
    tri = (jj < ii).astype(_BF)
    rank = jnp.dot(tri, mask.astype(_BF), preferred_element_type=_F32)
    r1 = jnp.sum(jnp.where(eid == a1, rank, 0.0), axis=-1, keepdims=True)
    r2 = jnp.sum(jnp.where(eid == a2, rank, 0.0), axis=-1, keepdims=True)
    rnk_ref[:, 0:1] = r1.astype(_I32)
    rnk_ref[:, 1:2] = r2.astype(_I32)
    cnt_ref[...] = jnp.sum(mask, axis=0, keepdims=True).astype(_I32)


def _router(x, ln_g, ln_b, router_W):
    return pl.pallas_call(
        _router_body,
        out_shape=[
            jax.ShapeDtypeStruct((S, H), _F32),
            jax.ShapeDtypeStruct((S, 2), _I32),
            jax.ShapeDtypeStruct((S, 2), _I32),
            jax.ShapeDtypeStruct((S, 2), _F32),
            jax.ShapeDtypeStruct((1, E), _I32),
            jax.ShapeDtypeStruct((1, 1), _F32),
            jax.ShapeDtypeStruct((1, 1), _F32),
        ],
    )(x, ln_g.reshape(1, H), ln_b.reshape(1, H), router_W)


def _sc_scatter_dispatch(xln, pkmaj):
    chunk = (2 * S) // _NW
    W = xln.shape[1]

    @functools.partial(
        pl.kernel, mesh=_sc_mesh(),
        out_type=jax.ShapeDtypeStruct((GPAD, W), xln.dtype),
        scratch_types=[pltpu.VMEM((chunk,), _I32),
                       pltpu.VMEM((chunk, W), xln.dtype),
                       pltpu.SemaphoreType.DMA],
    )
    def k(xln_h, pk_h, out_h, idx_v, rows_v, sem):
        wid = lax.axis_index("s") * _NC + lax.axis_index("c")
        astart = pl.multiple_of(wid * chunk, chunk)
        t0 = pl.multiple_of(jnp.bitwise_and(astart, S - 1), chunk)
        pltpu.sync_copy(pk_h.at[pl.ds(astart, chunk)], idx_v)
        pltpu.sync_copy(xln_h.at[pl.ds(t0, chunk)], rows_v)
        pltpu.async_copy(rows_v, out_h.at[idx_v], sem).wait()

    return k(xln, pkmaj)


def _sc_gather_rows(table, idxs):
    n = idxs.shape[0]
    W = table.shape[1]
    per_w = n // _NW
    chunk = per_w
    while chunk * W * 4 > 393216:
        chunk //= 2
    nch = per_w // chunk

    @functools.partial(
        pl.kernel, mesh=_sc_mesh(),
        out_type=jax.ShapeDtypeStruct((n, W), table.dtype),
        scratch_types=[pltpu.VMEM((chunk,), _I32),
                       pltpu.VMEM((chunk, W), table.dtype),
                       pltpu.SemaphoreType.DMA],
    )
    def k(tab_h, idx_h, out_h, idx_v, rows_v, sem):
        wid = lax.axis_index("s") * _NC + lax.axis_index("c")
        base = pl.multiple_of(wid * per_w, chunk)
        for c in range(nch):
            off = pl.multiple_of(base + c * chunk, chunk)
            pltpu.sync_copy(idx_h.at[pl.ds(off, chunk)], idx_v)
            pltpu.async_copy(tab_h.at[idx_v], rows_v, sem).wait()
            pltpu.sync_copy(rows_v, out_h.at[pl.ds(off, chunk)])

    return k(table, idxs)


def _ab_body(tm_ref, sm_ref, act_ref, xg_ref, wup_ref, wmid_ref, wd_ref,
             bup_ref, bh_ref, b1_ref, lg_ref, lbp_ref, bd_ref,
             out_ref, hs_ref):
    i = pl.program_id(0)
    e = tm_ref[i]
    m = lax.rem(e, 4)
    act = act_ref[i]

    @pl.when(act == 1)
    def _():
        xb = xg_ref[...].astype(_BF)
        h = _gelu(jnp.dot(xb, wup_ref[0], preferred_element_type=_F32)
                  + bup_ref[0])
        hs_ref[...] = h.astype(_BF)

    @pl.when((act == 1) & (m == 0))
    def _():
        t0 = (jnp.dot(hs_ref[...], wmid_ref[0], preferred_element_type=_F32)
              + b1_ref[0])
        t0 = _ln(t0, lg_ref[0], lbp_ref[0])
        hs_ref[...] = _gelu(t0).astype(_BF)

    @pl.when((act == 1) & (m == 1))
    def _():
        W = wmid_ref[0]
        h2 = _gelu(jnp.dot(hs_ref[...], W[:, :EH // 2],
                           preferred_element_type=_F32)
                   + bh_ref[0])
        h2b = h2.astype(_BF)
        sA = jnp.dot(h2b, W[:EH // 2, EH // 2:], preferred_element_type=_F32)
        sB = jnp.dot(h2b, W[EH // 2:, EH // 2:], preferred_element_type=_F32)
        s = jnp.concatenate([sA, sB], axis=1) + b1_ref[0]
        s = _ln(s, lg_ref[0], lbp_ref[0])
        hs_ref[...] = s.astype(_BF)

    @pl.when((act == 1) & (m == 2))
    def _():
        t0 = (jnp.dot(hs_ref[...], wmid_ref[0], preferred_element_type=_F32)
              + b1_ref[0])
        hs_ref[...] = _gelu(t0).astype(_BF)

    @pl.when(act == 1)
    def _():
        out_ref[...] = (jnp.dot(hs_ref[...], wd_ref[0],
                                preferred_element_type=_F32)
                        + bd_ref[0])


def _ab(xg, wup_all, wmid_all, wdown_all, bup_all, bh_all, b1_all, lg_all,
        lb_all, bd_all, tile_map, slot_map, active):
    grid_spec = pltpu.PrefetchScalarGridSpec(
        num_scalar_prefetch=3,
        grid=(NTILES,),
        in_specs=[
            pl.BlockSpec((T, H), lambda i, tm, sm, act: (i, 0)),
            pl.BlockSpec((1, H, EH), lambda i, tm, sm, act: (tm[i], 0, 0),
                         pipeline_mode=pl.Buffered(2)),
            pl.BlockSpec((1, EH, EH), lambda i, tm, sm, act: (sm[i], 0, 0),
                         pipeline_mode=pl.Buffered(2)),
            pl.BlockSpec((1, EH, H), lambda i, tm, sm, act: (tm[i], 0, 0),
                         pipeline_mode=pl.Buffered(1)),
            pl.BlockSpec((1, 1, EH), lambda i, tm, sm, act: (tm[i], 0, 0),
                         pipeline_mode=pl.Buffered(1)),
            pl.BlockSpec((1, 1, EH // 2), lambda i, tm, sm, act: (tm[i], 0, 0),
                         pipeline_mode=pl.Buffered(1)),
            pl.BlockSpec((1, 1, EH), lambda i, tm, sm, act: (tm[i], 0, 0),
                         pipeline_mode=pl.Buffered(1)),
            pl.BlockSpec((1, 1, EH), lambda i, tm, sm, act: (tm[i], 0, 0),
                         pipeline_mode=pl.Buffered(1)),
            pl.BlockSpec((1, 1, EH), lambda i, tm, sm, act: (tm[i], 0, 0),
                         pipeline_mode=pl.Buffered(1)),
            pl.BlockSpec((1, 1, H), lambda i, tm, sm, act: (tm[i], 0, 0),
                         pipeline_mode=pl.Buffered(1)),
        ],
        out_specs=pl.BlockSpec((T, H), lambda i, tm, sm, act: (i, 0)),
        scratch_shapes=[pltpu.VMEM((T, EH), _BF)],
    )
    return pl.pallas_call(
        _ab_body,
        grid_spec=grid_spec,
        out_shape=jax.ShapeDtypeStruct((GPAD, H), _F32),
    )(tile_map, slot_map, active, xg, wup_all, wmid_all, wdown_all,
      bup_all[:, None].astype(_BF), bh_all[:, None].astype(_BF),
      b1_all[:, None].astype(_BF), lg_all[:, None].astype(_BF),
      lb_all[:, None].astype(_BF), bd_all[:, None].astype(_BF))


def _comb_body(hid_ref, y0_ref, y1_ref, w_ref, out_ref):
    w = w_ref[...]
    out_ref[...] = (hid_ref[...] + y0_ref[...] * w[:, 0:1]
                    + y1_ref[...] * w[:, 1:2])


def _combine(x, yc, wgt2):
    return pl.pallas_call(
        _comb_body,
        grid=(S // T,),
        in_specs=[
            pl.BlockSpec((T, H), lambda i: (i, 0)),
            pl.BlockSpec((T, H), lambda i: (i, 0)),
            pl.BlockSpec((T, H), lambda i: (i + S // T, 0)),
            pl.BlockSpec((T, 2), lambda i: (i, 0)),
        ],
        out_specs=pl.BlockSpec((T, H), lambda i: (i, 0)),
        out_shape=jax.ShapeDtypeStruct((S, H), _F32),
    )(x, yc, yc, wgt2)


_RC = 384
_NCH = EH // _RC
_SQ_SLOTS = (0, 2, 3, 5)
_M1_SLOTS = (1, 4)


def _pack_mid_body(sq0_ref, sq1_ref, sq2_ref, sq3_ref,
                   a0_ref, b0_ref, a1_ref, b1_ref, out_ref):
    i = pl.program_id(0)
    slot = i // _NCH
    sqs = (sq0_ref, sq1_ref, sq2_ref, sq3_ref)
    for k, sl in enumerate(_SQ_SLOTS):
        @pl.when(slot == sl)
        def _(k=k):
            out_ref[...] = sqs[k][...].astype(_BF)
    abs_ = ((a0_ref, b0_ref), (a1_ref, b1_ref))
    for k, sl in enumerate(_M1_SLOTS):
        @pl.when(slot == sl)
        def _(k=k):
            a_ref, b_ref = abs_[k]
            out_ref[...] = jnp.concatenate(
                [a_ref[...], b_ref[...]], axis=1).astype(_BF)


def _pack_mid(ex):
    def sq_map(sl):
        return lambda i: (jnp.clip(i - sl * _NCH, 0, _NCH - 1), 0)

    def a_map(sl):
        return lambda i: (jnp.clip(i - sl * _NCH, 0, _NCH - 1), 0)

    def b_map(sl):
        def f(i):
            c = jnp.clip(i - sl * _NCH, 0, _NCH - 1)
            return (lax.rem(c, _NCH // 2), c // (_NCH // 2))
        return f

    out = pl.pallas_call(
        _pack_mid_body,
        grid=(6 * _NCH,),
        in_specs=[
            pl.BlockSpec((_RC, EH), sq_map(_SQ_SLOTS[0])),
            pl.BlockSpec((_RC, EH), sq_map(_SQ_SLOTS[1])),
            pl.BlockSpec((_RC, EH), sq_map(_SQ_SLOTS[2])),
            pl.BlockSpec((_RC, EH), sq_map(_SQ_SLOTS[3])),
            pl.BlockSpec((_RC, EH // 2), a_map(_M1_SLOTS[0]),
                         pipeline_mode=pl.Buffered(1)),
            pl.BlockSpec((_RC, EH // 2), b_map(_M1_SLOTS[0]),
                         pipeline_mode=pl.Buffered(1)),
            pl.BlockSpec((_RC, EH // 2), a_map(_M1_SLOTS[1]),
                         pipeline_mode=pl.Buffered(1)),
            pl.BlockSpec((_RC, EH // 2), b_map(_M1_SLOTS[1]),
                         pipeline_mode=pl.Buffered(1)),
        ],
        out_specs=pl.BlockSpec((_RC, EH), lambda i: (i, 0)),
        out_shape=jax.ShapeDtypeStruct((6 * EH, EH), _BF),
    )(ex[0]["Ws"], ex[2]["Ws"], ex[4]["Ws"], ex[6]["Ws"],
      ex[1]["Ws1"], ex[1]["Ws2"], ex[5]["Ws1"], ex[5]["Ws2"])
    return out.reshape(6, EH, EH)


def _pack_params(params):
    ex = params["experts"]
    wup_all = jnp.stack([p["W_up"] for p in ex]).astype(_BF)
    wdown_all = jnp.stack([p["W_down"] for p in ex]).astype(_BF)
    bup_all = jnp.stack([p["b_up"] for p in ex])
    bd_all = jnp.stack([p["b_down"] for p in ex])

    bh, b1, lg, lb = [], [], [], []
    zeh = jnp.zeros((EH,), _F32)
    oneh = jnp.ones((EH,), _F32)
    for i, p in enumerate(ex):
        m = i % 4
        if m in (0, 2):
            bh.append(jnp.zeros((EH // 2,), _F32))
            b1.append(p["bs"])
            lg.append(p.get("lng", oneh) if m == 0 else oneh)
            lb.append(p.get("lnb", zeh) if m == 0 else zeh)
        elif m == 1:
            bh.append(p["bs1"])
            b1.append(p["bs2"])
            lg.append(p["lng"])
            lb.append(p["lnb"])
        else:
            bh.append(jnp.zeros((EH // 2,), _F32))
            b1.append(zeh)
            lg.append(oneh)
            lb.append(zeh)
    wmid_all = _pack_mid(ex)
    bh_all = jnp.stack(bh)
    b1_all = jnp.stack(b1)
    lg_all = jnp.stack(lg)
    lb_all = jnp.stack(lb)
    return (wup_all, wmid_all, wdown_all, bup_all, bh_all, b1_all,
            lg_all, lb_all, bd_all)


def kernel(hidden_states, params):
    x = hidden_states.reshape(S, H)
    (wup_all, wmid_all, wdown_all, bup_all, bh_all, b1_all,
     lg_all, lb_all, bd_all) = _pack_params(params)

    xln, idx2, rnk2, wgt2, cnt, z, lb = _router(
        x, params["ln_g"], params["ln_b"], params["router_W"])

    counts = cnt[0]
    tiles_e = (counts + T - 1) // T
    cumt = jnp.cumsum(tiles_e)
    total_tiles = cumt[E - 1]
    poff = (jnp.concatenate([jnp.zeros((1,), _I32), cumt[:-1]]) * T).astype(_I32)
    tile_ids = jnp.arange(NTILES, dtype=_I32)
    tm_raw = jnp.searchsorted(cumt, tile_ids, side="right").astype(_I32)
    last_e = jnp.max(jnp.where(counts > 0,
                               jnp.arange(E, dtype=_I32), 0)).astype(_I32)
    tile_map = jnp.where(tile_ids < total_tiles, tm_raw, last_e)
    active = (tile_ids < total_tiles).astype(_I32)
    slot_of = jnp.array([0, 1, 2, 2, 3, 4, 5, 5], _I32)
    slot_map = slot_of[tile_map]

    pp = poff[idx2.reshape(-1)] + rnk2.reshape(-1)
    pkmaj = pp.reshape(S, 2).T.reshape(-1)

    xg = _sc_scatter_dispatch(xln, pkmaj)
    yg = _ab(xg, wup_all, wmid_all, wdown_all, bup_all, bh_all, b1_all,
             lg_all, lb_all, bd_all, tile_map, slot_map, active)
    yc = _sc_gather_rows(yg, pkmaj)
    out = _combine(x, yc, wgt2)

    return out.reshape(1, S, H), {"router_z_loss": z[0, 0],
                                  "load_balancing_loss": lb[0, 0]}


# speedup vs baseline: 1.0276x; 1.0276x over previous
import functools

import jax
import jax.numpy as jnp
from jax import lax
from jax.experimental import pallas as pl
from jax.experimental.pallas import tpu as pltpu
from jax.experimental.pallas import tpu_sc as plsc

H = 768
EH = 3072
E = 8
S = 2048
T = 256
NTILES = (2 * S) // T + E
GPAD = NTILES * T
Z_COEF = 0.001
LB_COEF = 0.01

_NC, _NS, _NW = 2, 16, 32


def _sc_mesh():
    return plsc.VectorSubcoreMesh(core_axis_name="c", subcore_axis_name="s")

_BF = jnp.bfloat16
_F32 = jnp.float32
_I32 = jnp.int32


def _gelu(x):
    return 0.5 * x * (1.0 + lax.erf(x * (2.0 ** -0.5)))


def _ln(x, g, b):
    m = jnp.mean(x, axis=-1, keepdims=True)
    v = jnp.mean((x - m) ** 2, axis=-1, keepdims=True)
    return (x - m) / jnp.sqrt(v + 1e-5) * g + b


def _router_body(x_ref, g_ref, b_ref, rw_ref, xln_ref, idx_ref, rnk_ref,
                 wgt_ref, cnt_ref, z_ref, lb_ref):
    x = x_ref[...]
    m = jnp.mean(x, axis=-1, keepdims=True)
    v = jnp.mean((x - m) ** 2, axis=-1, keepdims=True)
    xln = (x - m) / jnp.sqrt(v + 1e-5) * g_ref[...] + b_ref[...]
    xln_ref[...] = xln

    logits = jnp.dot(xln, rw_ref[...], preferred_element_type=_F32)
    mx = jnp.max(logits, axis=-1, keepdims=True)
    ex = jnp.exp(logits - mx)
    se = jnp.sum(ex, axis=-1, keepdims=True)
    lse = jnp.log(se) + mx
    z_ref[...] = jnp.mean(lse * lse).reshape(1, 1) * Z_COEF

    p = ex / se
    usage = jnp.mean(p, axis=0, keepdims=True)
    tgt = 1.0 / E
    lb_ref[...] = (jnp.sum(tgt * jnp.log(tgt / usage)) * LB_COEF).reshape(1, 1)

    eid = lax.broadcasted_iota(_I32, (S, E), 1)
    p1 = jnp.max(p, axis=-1, keepdims=True)
    a1 = jnp.min(jnp.where(p == p1, eid, E), axis=-1, keepdims=True)
    pm = jnp.where(eid == a1, -jnp.inf, p)
    p2 = jnp.max(pm, axis=-1, keepdims=True)
    a2 = jnp.min(jnp.where(pm == p2, eid, E), axis=-1, keepdims=True)
    s12 = p1 + p2
    wgt_ref[:, 0:1] = p1 / s12
    wgt_ref[:, 1:2] = p2 / s12
    idx_ref[:, 0:1] = a1
    idx_ref[:, 1:2] = a2

    mask = (eid == a1).astype(_F32) + (eid == a2).astype(_F32)
    ii = lax.broadcasted_iota(_I32, (S, S), 0)
    jj = lax.broadcasted_iota(_I32, (S, S), 1)
    tri = (jj < ii).astype(_BF)
    rank = jnp.dot(tri, mask.astype(_BF), preferred_element_type=_F32)
    r1 = jnp.sum(jnp.where(eid == a1, rank, 0.0), axis=-1, keepdims=True)
    r2 = jnp.sum(jnp.where(eid == a2, rank, 0.0), axis=-1, keepdims=True)
    rnk_ref[:, 0:1] = r1.astype(_I32)
    rnk_ref[:, 1:2] = r2.astype(_I32)
    cnt_ref[...] = jnp.sum(mask, axis=0, keepdims=True).astype(_I32)


def _router(x, ln_g, ln_b, router_W):
    return pl.pallas_call(
        _router_body,
        out_shape=[
            jax.ShapeDtypeStruct((S, H), _F32),
            jax.ShapeDtypeStruct((S, 2), _I32),
            jax.ShapeDtypeStruct((S, 2), _I32),
            jax.ShapeDtypeStruct((S, 2), _F32),
            jax.ShapeDtypeStruct((1, E), _I32),
            jax.ShapeDtypeStruct((1, 1), _F32),
            jax.ShapeDtypeStruct((1, 1), _F32),
        ],
    )(x, ln_g.reshape(1, H), ln_b.reshape(1, H), router_W)


def _sc_scatter_dispatch(xln, pkmaj):
    chunk = (2 * S) // _NW
    W = xln.shape[1]

    @functools.partial(
        pl.kernel, mesh=_sc_mesh(),
        out_type=jax.ShapeDtypeStruct((GPAD, W), xln.dtype),
        scratch_types=[pltpu.VMEM((chunk,), _I32),
                       pltpu.VMEM((chunk, W), xln.dtype),
                       pltpu.SemaphoreType.DMA],
    )
    def k(xln_h, pk_h, out_h, idx_v, rows_v, sem):
        wid = lax.axis_index("s") * _NC + lax.axis_index("c")
        astart = pl.multiple_of(wid * chunk, chunk)
        t0 = pl.multiple_of(jnp.bitwise_and(astart, S - 1), chunk)
        pltpu.sync_copy(pk_h.at[pl.ds(astart, chunk)], idx_v)
        pltpu.sync_copy(xln_h.at[pl.ds(t0, chunk)], rows_v)
        pltpu.async_copy(rows_v, out_h.at[idx_v], sem).wait()

    return k(xln, pkmaj)


def _sc_gather_rows(table, idxs):
    n = idxs.shape[0]
    W = table.shape[1]
    per_w = n // _NW
    chunk = per_w
    while chunk * W * 4 > 393216:
        chunk //= 2
    nch = per_w // chunk

    @functools.partial(
        pl.kernel, mesh=_sc_mesh(),
        out_type=jax.ShapeDtypeStruct((n, W), table.dtype),
        scratch_types=[pltpu.VMEM((chunk,), _I32),
                       pltpu.VMEM((chunk, W), table.dtype),
                       pltpu.SemaphoreType.DMA],
    )
    def k(tab_h, idx_h, out_h, idx_v, rows_v, sem):
        wid = lax.axis_index("s") * _NC + lax.axis_index("c")
        base = pl.multiple_of(wid * per_w, chunk)
        for c in range(nch):
            off = pl.multiple_of(base + c * chunk, chunk)
            pltpu.sync_copy(idx_h.at[pl.ds(off, chunk)], idx_v)
            pltpu.async_copy(tab_h.at[idx_v], rows_v, sem).wait()
            pltpu.sync_copy(rows_v, out_h.at[pl.ds(off, chunk)])

    return k(table, idxs)


def _ab_body(tm_ref, sm_ref, act_ref, xg_ref, wup_ref, wmid_ref, wd_ref,
             bup_ref, bh_ref, b1_ref, lg_ref, lbp_ref, bd_ref,
             out_ref, hs_ref):
    i = pl.program_id(0)
    e = tm_ref[i]
    m = lax.rem(e, 4)
    act = act_ref[i]

    @pl.when(act == 1)
    def _():
        xb = xg_ref[...].astype(_BF)
        h = _gelu(jnp.dot(xb, wup_ref[0], preferred_element_type=_F32)
                  + bup_ref[0])
        hs_ref[...] = h.astype(_BF)

    @pl.when((act == 1) & (m == 0))
    def _():
        t0 = (jnp.dot(hs_ref[...], wmid_ref[0], preferred_element_type=_F32)
              + b1_ref[0])
        t0 = _ln(t0, lg_ref[0], lbp_ref[0])
        hs_ref[...] = _gelu(t0).astype(_BF)

    @pl.when((act == 1) & (m == 1))
    def _():
        W = wmid_ref[0]
        h2 = _gelu(jnp.dot(hs_ref[...], W[:, :EH // 2],
                           preferred_element_type=_F32)
                   + bh_ref[0])
        h2b = h2.astype(_BF)
        sA = jnp.dot(h2b, W[:EH // 2, EH // 2:], preferred_element_type=_F32)
        sB = jnp.dot(h2b, W[EH // 2:, EH // 2:], preferred_element_type=_F32)
        s = jnp.concatenate([sA, sB], axis=1) + b1_ref[0]
        s = _ln(s, lg_ref[0], lbp_ref[0])
        hs_ref[...] = s.astype(_BF)

    @pl.when((act == 1) & (m == 2))
    def _():
        t0 = (jnp.dot(hs_ref[...], wmid_ref[0], preferred_element_type=_F32)
              + b1_ref[0])
        hs_ref[...] = _gelu(t0).astype(_BF)

    @pl.when(act == 1)
    def _():
        out_ref[...] = (jnp.dot(hs_ref[...], wd_ref[0],
                                preferred_element_type=_F32)
                        + bd_ref[0])


def _ab(xg, wup_all, wmid_all, wdown_all, bup_all, bh_all, b1_all, lg_all,
        lb_all, bd_all, tile_map, slot_map, active):
    grid_spec = pltpu.PrefetchScalarGridSpec(
        num_scalar_prefetch=3,
        grid=(NTILES,),
        in_specs=[
            pl.BlockSpec((T, H), lambda i, tm, sm, act: (i, 0)),
            pl.BlockSpec((1, H, EH), lambda i, tm, sm, act: (tm[i], 0, 0),
                         pipeline_mode=pl.Buffered(2)),
            pl.BlockSpec((1, EH, EH), lambda i, tm, sm, act: (sm[i], 0, 0),
                         pipeline_mode=pl.Buffered(2)),
            pl.BlockSpec((1, EH, H), lambda i, tm, sm, act: (tm[i], 0, 0),
                         pipeline_mode=pl.Buffered(1)),
            pl.BlockSpec((1, 1, EH), lambda i, tm, sm, act: (tm[i], 0, 0),
                         pipeline_mode=pl.Buffered(1)),
            pl.BlockSpec((1, 1, EH // 2), lambda i, tm, sm, act: (tm[i], 0, 0),
                         pipeline_mode=pl.Buffered(1)),
            pl.BlockSpec((1, 1, EH), lambda i, tm, sm, act: (tm[i], 0, 0),
                         pipeline_mode=pl.Buffered(1)),
            pl.BlockSpec((1, 1, EH), lambda i, tm, sm, act: (tm[i], 0, 0),
                         pipeline_mode=pl.Buffered(1)),
            pl.BlockSpec((1, 1, EH), lambda i, tm, sm, act: (tm[i], 0, 0),
                         pipeline_mode=pl.Buffered(1)),
            pl.BlockSpec((1, 1, H), lambda i, tm, sm, act: (tm[i], 0, 0),
                         pipeline_mode=pl.Buffered(1)),
        ],
        out_specs=pl.BlockSpec((T, H), lambda i, tm, sm, act: (i, 0)),
        scratch_shapes=[pltpu.VMEM((T, EH), _BF)],
    )
    return pl.pallas_call(
        _ab_body,
        grid_spec=grid_spec,
        out_shape=jax.ShapeDtypeStruct((GPAD, H), _F32),
    )(tile_map, slot_map, active, xg, wup_all, wmid_all, wdown_all,
      bup_all[:, None].astype(_BF), bh_all[:, None].astype(_BF),
      b1_all[:, None].astype(_BF), lg_all[:, None].astype(_BF),
      lb_all[:, None].astype(_BF), bd_all[:, None].astype(_BF))


def _comb_body(hid_ref, y0_ref, y1_ref, w_ref, out_ref):
    w = w_ref[...]
    out_ref[...] = (hid_ref[...] + y0_ref[...] * w[:, 0:1]
                    + y1_ref[...] * w[:, 1:2])


def _combine(x, yc, wgt2):
    return pl.pallas_call(
        _comb_body,
        grid=(S // T,),
        in_specs=[
            pl.BlockSpec((T, H), lambda i: (i, 0)),
            pl.BlockSpec((T, H), lambda i: (i, 0)),
            pl.BlockSpec((T, H), lambda i: (i + S // T, 0)),
            pl.BlockSpec((T, 2), lambda i: (i, 0)),
        ],
        out_specs=pl.BlockSpec((T, H), lambda i: (i, 0)),
        out_shape=jax.ShapeDtypeStruct((S, H), _F32),
    )(x, yc, yc, wgt2)


_RC = 256
_NCH = EH // _RC
_SQ_SLOTS = (0, 2, 3, 5)
_M1_SLOTS = (1, 4)


def _pack_mid_body(sq0_ref, sq1_ref, sq2_ref, sq3_ref,
                   a0_ref, b0_ref, a1_ref, b1_ref, out_ref):
    i = pl.program_id(0)
    slot = i // _NCH
    sqs = (sq0_ref, sq1_ref, sq2_ref, sq3_ref)
    for k, sl in enumerate(_SQ_SLOTS):
        @pl.when(slot == sl)
        def _(k=k):
            out_ref[...] = sqs[k][...].astype(_BF)
    abs_ = ((a0_ref, b0_ref), (a1_ref, b1_ref))
    for k, sl in enumerate(_M1_SLOTS):
        @pl.when(slot == sl)
        def _(k=k):
            a_ref, b_ref = abs_[k]
            out_ref[...] = jnp.concatenate(
                [a_ref[...], b_ref[...]], axis=1).astype(_BF)


def _pack_mid(ex):
    def sq_map(sl):
        return lambda i: (jnp.clip(i - sl * _NCH, 0, _NCH - 1), 0)

    def a_map(sl):
        return lambda i: (jnp.clip(i - sl * _NCH, 0, _NCH - 1), 0)

    def b_map(sl):
        def f(i):
            c = jnp.clip(i - sl * _NCH, 0, _NCH - 1)
            return (lax.rem(c, _NCH // 2), c // (_NCH // 2))
        return f

    out = pl.pallas_call(
        _pack_mid_body,
        grid=(6 * _NCH,),
        in_specs=[
            pl.BlockSpec((_RC, EH), sq_map(_SQ_SLOTS[0])),
            pl.BlockSpec((_RC, EH), sq_map(_SQ_SLOTS[1])),
            pl.BlockSpec((_RC, EH), sq_map(_SQ_SLOTS[2])),
            pl.BlockSpec((_RC, EH), sq_map(_SQ_SLOTS[3])),
            pl.BlockSpec((_RC, EH // 2), a_map(_M1_SLOTS[0])),
            pl.BlockSpec((_RC, EH // 2), b_map(_M1_SLOTS[0])),
            pl.BlockSpec((_RC, EH // 2), a_map(_M1_SLOTS[1])),
            pl.BlockSpec((_RC, EH // 2), b_map(_M1_SLOTS[1])),
        ],
        out_specs=pl.BlockSpec((_RC, EH), lambda i: (i, 0)),
        out_shape=jax.ShapeDtypeStruct((6 * EH, EH), _BF),
    )(ex[0]["Ws"], ex[2]["Ws"], ex[4]["Ws"], ex[6]["Ws"],
      ex[1]["Ws1"], ex[1]["Ws2"], ex[5]["Ws1"], ex[5]["Ws2"])
    return out.reshape(6, EH, EH)


def _pack_params(params):
    ex = params["experts"]
    wup_all = jnp.stack([p["W_up"] for p in ex]).astype(_BF)
    wdown_all = jnp.stack([p["W_down"] for p in ex]).astype(_BF)
    bup_all = jnp.stack([p["b_up"] for p in ex])
    bd_all = jnp.stack([p["b_down"] for p in ex])

    bh, b1, lg, lb = [], [], [], []
    zeh = jnp.zeros((EH,), _F32)
    oneh = jnp.ones((EH,), _F32)
    for i, p in enumerate(ex):
        m = i % 4
        if m in (0, 2):
            bh.append(jnp.zeros((EH // 2,), _F32))
            b1.append(p["bs"])
            lg.append(p.get("lng", oneh) if m == 0 else oneh)
            lb.append(p.get("lnb", zeh) if m == 0 else zeh)
        elif m == 1:
            bh.append(p["bs1"])
            b1.append(p["bs2"])
            lg.append(p["lng"])
            lb.append(p["lnb"])
        else:
            bh.append(jnp.zeros((EH // 2,), _F32))
            b1.append(zeh)
            lg.append(oneh)
            lb.append(zeh)
    wmid_all = _pack_mid(ex)
    bh_all = jnp.stack(bh)
    b1_all = jnp.stack(b1)
    lg_all = jnp.stack(lg)
    lb_all = jnp.stack(lb)
    return (wup_all, wmid_all, wdown_all, bup_all, bh_all, b1_all,
            lg_all, lb_all, bd_all)


def kernel(hidden_states, params):
    x = hidden_states.reshape(S, H)
    (wup_all, wmid_all, wdown_all, bup_all, bh_all, b1_all,
     lg_all, lb_all, bd_all) = _pack_params(params)

    xln, idx2, rnk2, wgt2, cnt, z, lb = _router(
        x, params["ln_g"], params["ln_b"], params["router_W"])

    counts = cnt[0]
    tiles_e = (counts + T - 1) // T
    cumt = jnp.cumsum(tiles_e)
    total_tiles = cumt[E - 1]
    poff = (jnp.concatenate([jnp.zeros((1,), _I32), cumt[:-1]]) * T).astype(_I32)
    tile_ids = jnp.arange(NTILES, dtype=_I32)
    tm_raw = jnp.searchsorted(cumt, tile_ids, side="right").astype(_I32)
    last_e = jnp.max(jnp.where(counts > 0,
                               jnp.arange(E, dtype=_I32), 0)).astype(_I32)
    tile_map = jnp.where(tile_ids < total_tiles, tm_raw, last_e)
    active = (tile_ids < total_tiles).astype(_I32)
    slot_of = jnp.array([0, 1, 2, 2, 3, 4, 5, 5], _I32)
    slot_map = slot_of[tile_map]

    pp = poff[idx2.reshape(-1)] + rnk2.reshape(-1)
    pkmaj = pp.reshape(S, 2).T.reshape(-1)

    xg = _sc_scatter_dispatch(xln, pkmaj)
    yg = _ab(xg, wup_all, wmid_all, wdown_all, bup_all, bh_all, b1_all,
             lg_all, lb_all, bd_all, tile_map, slot_map, active)
    yc = _sc_gather_rows(yg, pkmaj)
    out = _combine(x, yc, wgt2)

    return out.reshape(1, S, H), {"router_z_loss": z[0, 0],
                                  "load_balancing_loss": lb[0, 0]}
